# Initial kernel scaffold; baseline (speedup 1.0000x reference)
import functools

import jax
import jax.numpy as jnp
from jax import lax
from jax.experimental import pallas as pl
from jax.experimental.pallas import tpu as pltpu
from jax.experimental.pallas import tpu_sc as plsc

B = 16384
D_EMB = 32
D_DRUG = 64
D_LIN = 32

NC = 2
NS = 16
NW = NC * NS
BPW = B // NW
ICH = 128
NCH = BPW // ICH
G16 = BPW // 16


def _fixup_zero_rows(idx_v, rows_v, D):
    zeros = jnp.zeros((16,), jnp.float32)

    def body(g, carry):
        gb = g * 16
        idx16 = idx_v[pl.ds(gb, 16)]
        iszero = idx16 == 0
        anyz = jnp.max(iszero.astype(jnp.int32))

        @pl.when(anyz > 0)
        def _():
            rowi = gb + lax.iota(jnp.int32, 16)
            for c in range(D):
                coli = jnp.full((16,), c, jnp.int32)
                plsc.store_scatter(rows_v, [rowi, coli], zeros, mask=iszero)

        return carry

    lax.fori_loop(0, G16, body, 0)


def _make_gather4():
    mesh = plsc.VectorSubcoreMesh(core_axis_name="c", subcore_axis_name="s")

    @functools.partial(
        pl.kernel,
        mesh=mesh,
        out_type=(
            jax.ShapeDtypeStruct((B, D_EMB), jnp.float32),
            jax.ShapeDtypeStruct((B, D_DRUG), jnp.float32),
            jax.ShapeDtypeStruct((B, D_EMB), jnp.float32),
            jax.ShapeDtypeStruct((B, D_EMB), jnp.float32),
        ),
        scratch_types=[
            pltpu.VMEM((NCH, ICH), jnp.int32),
            pltpu.VMEM((NCH, ICH), jnp.int32),
            pltpu.VMEM((NCH, ICH), jnp.int32),
            pltpu.VMEM((NCH, ICH), jnp.int32),
            pltpu.VMEM((BPW, D_EMB), jnp.float32),
            pltpu.VMEM((BPW, D_DRUG), jnp.float32),
            pltpu.VMEM((BPW, D_EMB), jnp.float32),
            pltpu.VMEM((BPW, D_EMB), jnp.float32),
            pltpu.VMEM((BPW,), jnp.int32),
            pltpu.VMEM((BPW,), jnp.int32),
            pltpu.VMEM((BPW,), jnp.int32),
            pltpu.VMEM((BPW,), jnp.int32),
            pltpu.SemaphoreType.DMA,
            pltpu.SemaphoreType.DMA,
            pltpu.SemaphoreType.DMA,
            pltpu.SemaphoreType.DMA,
        ],
    )
    def gather4(mt_h, dn_h, cs_h, ns_h, wm_h, wd_h, wc_h, wn_h,
                om_h, od_h, oc_h, on_h,
                im_v, id_v, ic_v, in_v,
                rm_v, rd_v, rc_v, rn_v,
                fm_v, fd_v, fc_v, fn_v,
                sem_m, sem_d, sem_c, sem_n):
        wid = lax.axis_index("s") * NC + lax.axis_index("c")
        base = wid * BPW

        for ih, iv, fv in ((mt_h, im_v, fm_v), (dn_h, id_v, fd_v),
                           (cs_h, ic_v, fc_v), (ns_h, in_v, fn_v)):
            pltpu.sync_copy(ih.at[pl.ds(base, BPW)], fv)
            for j in range(NCH):
                pltpu.sync_copy(ih.at[pl.ds(base + j * ICH, ICH)], iv.at[j])

        copies = []
        for wh, iv, rv, sem in ((wm_h, im_v, rm_v, sem_m),
                                (wd_h, id_v, rd_v, sem_d),
                                (wc_h, ic_v, rc_v, sem_c),
                                (wn_h, in_v, rn_v, sem_n)):
            for j in range(NCH):
                copies.append(pltpu.async_copy(
                    wh.at[iv.at[j]], rv.at[pl.ds(j * ICH, ICH)], sem))
        for cp in copies:
            cp.wait()

        _fixup_zero_rows(fm_v, rm_v, D_EMB)
        _fixup_zero_rows(fd_v, rd_v, D_DRUG)
        _fixup_zero_rows(fc_v, rc_v, D_EMB)
        _fixup_zero_rows(fn_v, rn_v, D_EMB)

        pltpu.sync_copy(rm_v, om_h.at[pl.ds(base, BPW)])
        pltpu.sync_copy(rd_v, od_h.at[pl.ds(base, BPW)])
        pltpu.sync_copy(rc_v, oc_h.at[pl.ds(base, BPW)])
        pltpu.sync_copy(rn_v, on_h.at[pl.ds(base, BPW)])

    return gather4


_gather4 = _make_gather4()


LIN_BLK = 2048


def _lin_body(x1, w1, b1, x2, w2, b2, x3, w3, b3, x4, w4, b4, x5, w5, b5,
              o1, o2, o3, o4, o5):
    for x, w, b, o in ((x1, w1, b1, o1), (x2, w2, b2, o2), (x3, w3, b3, o3),
                       (x4, w4, b4, o4), (x5, w5, b5, o5)):
        o[...] = x[...] * w[...] + b[...]


def _lin5(xs, ws, bs):
    x_spec = pl.BlockSpec((LIN_BLK, 1), lambda i: (i, 0))
    w_spec = pl.BlockSpec((1, D_LIN), lambda i: (0, 0))
    o_spec = pl.BlockSpec((LIN_BLK, D_LIN), lambda i: (i, 0))
    in_specs = []
    operands = []
    for x, w, b in zip(xs, ws, bs):
        in_specs += [x_spec, w_spec, w_spec]
        operands += [x, w.reshape(1, D_LIN), b.reshape(1, D_LIN)]
    return pl.pallas_call(
        _lin_body,
        grid=(B // LIN_BLK,),
        in_specs=in_specs,
        out_specs=[o_spec] * 5,
        out_shape=[jax.ShapeDtypeStruct((B, D_LIN), jnp.float32)] * 5,
    )(*operands)


def kernel(media_type, temperature, pre_culture_time, pre_culture_od600,
           drug_culture_time, drug_name, concentration, carbon_source,
           nitrogen_source, W_media, W_drug, W_carbon, W_nitrogen,
           W_temp, b_temp, W_pct, b_pct, W_od, b_od, W_dct, b_dct,
           W_conc, b_conc):
    mt = media_type.astype(jnp.int32)
    dn = drug_name.astype(jnp.int32)
    cs = carbon_source.astype(jnp.int32)
    ns_ = nitrogen_source.astype(jnp.int32)

    emb_media, emb_drug, emb_carbon, emb_nitro = _gather4(
        mt, dn, cs, ns_, W_media, W_drug, W_carbon, W_nitrogen)

    lt, lpct, lod, ldct, lconc = _lin5(
        (temperature, pre_culture_time, pre_culture_od600, drug_culture_time,
         concentration),
        (W_temp, W_pct, W_od, W_dct, W_conc),
        (b_temp, b_pct, b_od, b_dct, b_conc))

    return (emb_media, lt, lpct, lod, ldct, emb_drug, lconc, emb_carbon,
            emb_nitro)


# speedup vs baseline: 1.2991x; 1.2991x over previous
import functools

import jax
import jax.numpy as jnp
from jax import lax
from jax.experimental import pallas as pl
from jax.experimental.pallas import tpu as pltpu
from jax.experimental.pallas import tpu_sc as plsc

B = 16384
D_EMB = 32
D_DRUG = 64
D_LIN = 32

NC = 2
NS = 16
NW = NC * NS
BPW = B // NW
ICH = 128
NCH = BPW // ICH
G16 = BPW // 16


def _fixup_zero_rows(idx_v, rows_v, D):
    zeros = jnp.zeros((16,), jnp.float32)

    def body(g, carry):
        gb = g * 16
        idx16 = idx_v[pl.ds(gb, 16)]
        minv = jnp.min(idx16)

        @pl.when(minv == 0)
        def _():
            iszero = idx16 == 0
            rowi = gb + lax.iota(jnp.int32, 16)
            for c in range(D):
                coli = jnp.full((16,), c, jnp.int32)
                plsc.store_scatter(rows_v, [rowi, coli], zeros, mask=iszero)

        return carry

    lax.fori_loop(0, G16, body, 0)


FIXUP = True


def _make_gather4():
    mesh = plsc.VectorSubcoreMesh(core_axis_name="c", subcore_axis_name="s",
                                  num_cores=NC, num_subcores=NS)

    @functools.partial(
        pl.kernel,
        mesh=mesh,
        compiler_params=pltpu.CompilerParams(use_tc_tiling_on_sc=False,
                                             needs_layout_passes=False),
        out_type=(
            jax.ShapeDtypeStruct((B, D_EMB), jnp.float32),
            jax.ShapeDtypeStruct((B, D_DRUG), jnp.float32),
            jax.ShapeDtypeStruct((B, D_EMB), jnp.float32),
            jax.ShapeDtypeStruct((B, D_EMB), jnp.float32),
        ),
        scratch_types=[
            pltpu.VMEM((NCH, ICH), jnp.int32),
            pltpu.VMEM((NCH, ICH), jnp.int32),
            pltpu.VMEM((NCH, ICH), jnp.int32),
            pltpu.VMEM((NCH, ICH), jnp.int32),
            pltpu.VMEM((BPW, D_EMB), jnp.float32),
            pltpu.VMEM((BPW, D_DRUG), jnp.float32),
            pltpu.VMEM((BPW, D_EMB), jnp.float32),
            pltpu.VMEM((BPW, D_EMB), jnp.float32),
            pltpu.VMEM((BPW,), jnp.int32),
            pltpu.VMEM((BPW,), jnp.int32),
            pltpu.VMEM((BPW,), jnp.int32),
            pltpu.VMEM((BPW,), jnp.int32),
            pltpu.SemaphoreType.DMA,
            pltpu.SemaphoreType.DMA,
            pltpu.SemaphoreType.DMA,
            pltpu.SemaphoreType.DMA,
        ],
    )
    def gather4(mt_h, dn_h, cs_h, ns_h, wm_h, wd_h, wc_h, wn_h,
                om_h, od_h, oc_h, on_h,
                im_v, id_v, ic_v, in_v,
                rm_v, rd_v, rc_v, rn_v,
                fm_v, fd_v, fc_v, fn_v,
                sem_m, sem_d, sem_c, sem_n):
        wid = lax.axis_index("s") * NC + lax.axis_index("c")
        base = wid * BPW

        for ih, iv, fv in ((mt_h, im_v, fm_v), (dn_h, id_v, fd_v),
                           (cs_h, ic_v, fc_v), (ns_h, in_v, fn_v)):
            pltpu.sync_copy(ih.at[pl.ds(base, BPW)], fv)
            for j in range(NCH):
                pltpu.sync_copy(ih.at[pl.ds(base + j * ICH, ICH)], iv.at[j])

        copies = []
        for wh, iv, rv, sem in ((wm_h, im_v, rm_v, sem_m),
                                (wd_h, id_v, rd_v, sem_d),
                                (wc_h, ic_v, rc_v, sem_c),
                                (wn_h, in_v, rn_v, sem_n)):
            for j in range(NCH):
                copies.append(pltpu.async_copy(
                    wh.at[iv.at[j]], rv.at[pl.ds(j * ICH, ICH)], sem))
        for cp in copies:
            cp.wait()

        if FIXUP:
            _fixup_zero_rows(fm_v, rm_v, D_EMB)
            _fixup_zero_rows(fd_v, rd_v, D_DRUG)
            _fixup_zero_rows(fc_v, rc_v, D_EMB)
            _fixup_zero_rows(fn_v, rn_v, D_EMB)

        pltpu.sync_copy(rm_v, om_h.at[pl.ds(base, BPW)])
        pltpu.sync_copy(rd_v, od_h.at[pl.ds(base, BPW)])
        pltpu.sync_copy(rc_v, oc_h.at[pl.ds(base, BPW)])
        pltpu.sync_copy(rn_v, on_h.at[pl.ds(base, BPW)])

    return gather4


_gather4 = _make_gather4()


LIN_BLK = 2048


def _lin_body(x1, w1, b1, x2, w2, b2, x3, w3, b3, x4, w4, b4, x5, w5, b5,
              o1, o2, o3, o4, o5):
    for x, w, b, o in ((x1, w1, b1, o1), (x2, w2, b2, o2), (x3, w3, b3, o3),
                       (x4, w4, b4, o4), (x5, w5, b5, o5)):
        o[...] = x[...] * w[...] + b[...]


def _lin5(xs, ws, bs):
    x_spec = pl.BlockSpec((LIN_BLK, 1), lambda i: (i, 0))
    w_spec = pl.BlockSpec((1, D_LIN), lambda i: (0, 0))
    o_spec = pl.BlockSpec((LIN_BLK, D_LIN), lambda i: (i, 0))
    in_specs = []
    operands = []
    for x, w, b in zip(xs, ws, bs):
        in_specs += [x_spec, w_spec, w_spec]
        operands += [x, w.reshape(1, D_LIN), b.reshape(1, D_LIN)]
    return pl.pallas_call(
        _lin_body,
        grid=(B // LIN_BLK,),
        in_specs=in_specs,
        out_specs=[o_spec] * 5,
        out_shape=[jax.ShapeDtypeStruct((B, D_LIN), jnp.float32)] * 5,
    )(*operands)


def kernel(media_type, temperature, pre_culture_time, pre_culture_od600,
           drug_culture_time, drug_name, concentration, carbon_source,
           nitrogen_source, W_media, W_drug, W_carbon, W_nitrogen,
           W_temp, b_temp, W_pct, b_pct, W_od, b_od, W_dct, b_dct,
           W_conc, b_conc):
    mt = media_type.astype(jnp.int32)
    dn = drug_name.astype(jnp.int32)
    cs = carbon_source.astype(jnp.int32)
    ns_ = nitrogen_source.astype(jnp.int32)

    emb_media, emb_drug, emb_carbon, emb_nitro = _gather4(
        mt, dn, cs, ns_, W_media, W_drug, W_carbon, W_nitrogen)

    lt, lpct, lod, ldct, lconc = _lin5(
        (temperature, pre_culture_time, pre_culture_od600, drug_culture_time,
         concentration),
        (W_temp, W_pct, W_od, W_dct, W_conc),
        (b_temp, b_pct, b_od, b_dct, b_conc))

    return (emb_media, lt, lpct, lod, ldct, emb_drug, lconc, emb_carbon,
            emb_nitro)


# speedup vs baseline: 2.7421x; 2.1107x over previous
import functools

import jax
import jax.numpy as jnp
from jax import lax
from jax.experimental import pallas as pl
from jax.experimental.pallas import tpu as pltpu
from jax.experimental.pallas import tpu_sc as plsc

B = 16384
V_SMALL = 1000
V_DRUG = 100000
D_EMB = 32
D_DRUG = 64
D_LIN = 32

NC = 2
NS = 16
NW = NC * NS
CH = 2048
NCHK = B // CH
UNROLL = 4
GROUPS = CH // 16


def _gather_chunk(row_v, idx_v, out_v):
    zero = jnp.zeros((16,), jnp.float32)

    def body(g, carry):
        base = g * (16 * UNROLL)
        for u in range(UNROLL):
            off = base + u * 16
            idx16 = idx_v[pl.ds(off, 16)]
            v = plsc.load_gather(row_v, [idx16])
            v = jnp.where(idx16 != 0, v, zero)
            out_v[pl.ds(off, 16)] = v
        return carry

    lax.fori_loop(0, GROUPS // UNROLL, body, 0)


def _make_gather4():
    mesh = plsc.VectorSubcoreMesh(core_axis_name="c", subcore_axis_name="s",
                                  num_cores=NC, num_subcores=NS)

    @functools.partial(
        pl.kernel,
        mesh=mesh,
        compiler_params=pltpu.CompilerParams(needs_layout_passes=False),
        out_type=(
            jax.ShapeDtypeStruct((D_EMB, B), jnp.float32),
            jax.ShapeDtypeStruct((D_DRUG, B), jnp.float32),
            jax.ShapeDtypeStruct((D_EMB, B), jnp.float32),
            jax.ShapeDtypeStruct((D_EMB, B), jnp.float32),
        ),
        scratch_types=[
            pltpu.VMEM((V_SMALL,), jnp.float32),
            pltpu.VMEM((V_DRUG,), jnp.float32),
            pltpu.VMEM((V_SMALL,), jnp.float32),
            pltpu.VMEM((V_SMALL,), jnp.float32),
            pltpu.VMEM((CH,), jnp.int32),
            pltpu.VMEM((CH,), jnp.int32),
            pltpu.VMEM((CH,), jnp.int32),
            pltpu.VMEM((CH,), jnp.int32),
            pltpu.VMEM((CH,), jnp.float32),
            pltpu.VMEM((CH,), jnp.float32),
            pltpu.VMEM((CH,), jnp.float32),
            pltpu.VMEM((CH,), jnp.float32),
        ],
    )
    def gather4(mt_h, dn_h, cs_h, ns_h, wmt_h, wdt_h, wct_h, wnt_h,
                omt_h, odt_h, oct_h, ont_h,
                row_m, row_d, row_c, row_n,
                idx_m, idx_d, idx_c, idx_n,
                out_m, out_d, out_c, out_n):
        w = lax.axis_index("s") * NC + lax.axis_index("c")

        pltpu.sync_copy(wmt_h.at[w], row_m)
        pltpu.sync_copy(wct_h.at[w], row_c)
        pltpu.sync_copy(wnt_h.at[w], row_n)
        for ck in range(NCHK):
            base = ck * CH
            pltpu.sync_copy(mt_h.at[pl.ds(base, CH)], idx_m)
            pltpu.sync_copy(cs_h.at[pl.ds(base, CH)], idx_c)
            pltpu.sync_copy(ns_h.at[pl.ds(base, CH)], idx_n)
            _gather_chunk(row_m, idx_m, out_m)
            _gather_chunk(row_c, idx_c, out_c)
            _gather_chunk(row_n, idx_n, out_n)
            pltpu.sync_copy(out_m, omt_h.at[w, pl.ds(base, CH)])
            pltpu.sync_copy(out_c, oct_h.at[w, pl.ds(base, CH)])
            pltpu.sync_copy(out_n, ont_h.at[w, pl.ds(base, CH)])

        for half in range(2):
            dr = w + half * NW
            pltpu.sync_copy(wdt_h.at[dr], row_d)
            for ck in range(NCHK):
                base = ck * CH
                pltpu.sync_copy(dn_h.at[pl.ds(base, CH)], idx_d)
                _gather_chunk(row_d, idx_d, out_d)
                pltpu.sync_copy(out_d, odt_h.at[dr, pl.ds(base, CH)])

    return gather4


_gather4 = _make_gather4()


LIN_BLK = 2048


def _lin_body(x1, x2, x3, x4, x5, w_ref, b_ref, o1, o2, o3, o4, o5):
    for k, (x, o) in enumerate(((x1, o1), (x2, o2), (x3, o3), (x4, o4),
                                (x5, o5))):
        o[...] = w_ref[k] * x[...][None, :] + b_ref[k]


def _lin5(xs, ws, bs):
    x_spec = pl.BlockSpec((LIN_BLK,), lambda i: (i,))
    wb_spec = pl.BlockSpec((5, D_LIN, 1), lambda i: (0, 0, 0))
    o_spec = pl.BlockSpec((D_LIN, LIN_BLK), lambda i: (0, i))
    w5 = jnp.stack([w.reshape(D_LIN) for w in ws])[:, :, None]
    b5 = jnp.stack([b.reshape(D_LIN) for b in bs])[:, :, None]
    outs = pl.pallas_call(
        _lin_body,
        grid=(B // LIN_BLK,),
        in_specs=[x_spec] * 5 + [wb_spec, wb_spec],
        out_specs=[o_spec] * 5,
        out_shape=[jax.ShapeDtypeStruct((D_LIN, B), jnp.float32)] * 5,
    )(*[x.reshape(B) for x in xs], w5, b5)
    return [o.T for o in outs]


def kernel(media_type, temperature, pre_culture_time, pre_culture_od600,
           drug_culture_time, drug_name, concentration, carbon_source,
           nitrogen_source, W_media, W_drug, W_carbon, W_nitrogen,
           W_temp, b_temp, W_pct, b_pct, W_od, b_od, W_dct, b_dct,
           W_conc, b_conc):
    mt = media_type.astype(jnp.int32)
    dn = drug_name.astype(jnp.int32)
    cs = carbon_source.astype(jnp.int32)
    ns_ = nitrogen_source.astype(jnp.int32)

    omt, odt, oct_, ont = _gather4(
        mt, dn, cs, ns_, W_media.T, W_drug.T, W_carbon.T, W_nitrogen.T)

    lt, lpct, lod, ldct, lconc = _lin5(
        (temperature, pre_culture_time, pre_culture_od600, drug_culture_time,
         concentration),
        (W_temp, W_pct, W_od, W_dct, W_conc),
        (b_temp, b_pct, b_od, b_dct, b_conc))

    return (omt.T, lt, lpct, lod, ldct, odt.T, lconc, oct_.T, ont.T)


# speedup vs baseline: 4.0985x; 1.4946x over previous
import functools

import jax
import jax.numpy as jnp
from jax import lax
from jax.experimental import pallas as pl
from jax.experimental.pallas import tpu as pltpu
from jax.experimental.pallas import tpu_sc as plsc

B = 16384
V_SMALL = 1000
V_DRUG = 100000
D_EMB = 32
D_DRUG = 64
D_LIN = 32

NC = 2
NS = 16
NW = NC * NS
CH = 2048
NCHK = B // CH
UNROLL = 4
GROUPS = CH // 16


def _gather_chunk(rows, idxs, outs):
    zero = jnp.zeros((16,), jnp.float32)

    def body(g, carry):
        base = g * (16 * UNROLL)
        for u in range(UNROLL):
            off = base + u * 16
            for row_v, idx_v, out_v in zip(rows, idxs, outs):
                idx16 = idx_v[pl.ds(off, 16)]
                v = plsc.load_gather(row_v, [idx16])
                out_v[pl.ds(off, 16)] = jnp.where(idx16 != 0, v, zero)
        return carry

    lax.fori_loop(0, GROUPS // UNROLL, body, 0)


def _make_gather4():
    mesh = plsc.VectorSubcoreMesh(core_axis_name="c", subcore_axis_name="s",
                                  num_cores=NC, num_subcores=NS)

    @functools.partial(
        pl.kernel,
        mesh=mesh,
        compiler_params=pltpu.CompilerParams(needs_layout_passes=False),
        out_type=(
            jax.ShapeDtypeStruct((D_EMB, B), jnp.float32),
            jax.ShapeDtypeStruct((D_DRUG, B), jnp.float32),
            jax.ShapeDtypeStruct((D_EMB, B), jnp.float32),
            jax.ShapeDtypeStruct((D_EMB, B), jnp.float32),
        ),
        scratch_types=[
            pltpu.VMEM((V_SMALL,), jnp.float32),
            pltpu.VMEM((V_DRUG,), jnp.float32),
            pltpu.VMEM((V_SMALL,), jnp.float32),
            pltpu.VMEM((V_SMALL,), jnp.float32),
            pltpu.VMEM((CH,), jnp.int32),
            pltpu.VMEM((CH,), jnp.int32),
            pltpu.VMEM((CH,), jnp.int32),
            pltpu.VMEM((CH,), jnp.int32),
            pltpu.VMEM((CH,), jnp.int32),
            pltpu.VMEM((CH,), jnp.int32),
            pltpu.VMEM((CH,), jnp.float32),
            pltpu.VMEM((CH,), jnp.float32),
            pltpu.VMEM((CH,), jnp.float32),
            pltpu.VMEM((CH,), jnp.float32),
            pltpu.VMEM((CH,), jnp.float32),
            pltpu.VMEM((CH,), jnp.float32),
            pltpu.SemaphoreType.DMA,
            pltpu.SemaphoreType.DMA,
            pltpu.SemaphoreType.DMA,
        ],
    )
    def gather4(mt_h, dn_h, cs_h, ns_h, wmt_h, wdt_h, wct_h, wnt_h,
                omt_h, odt_h, oct_h, ont_h,
                row_m, row_d, row_c, row_n,
                ib_m0, ib_m1, ib_c0, ib_c1, ib_n0, ib_n1,
                ob_m0, ob_m1, ob_c0, ob_c1, ob_n0, ob_n1,
                sem_row, sem_in, sem_out):
        w = lax.axis_index("s") * NC + lax.axis_index("c")

        rows = (row_m, row_c, row_n)
        idx_hs = (mt_h, cs_h, ns_h)
        out_hs = (omt_h, oct_h, ont_h)
        ibufs = ((ib_m0, ib_m1), (ib_c0, ib_c1), (ib_n0, ib_n1))
        obufs = ((ob_m0, ob_m1), (ob_c0, ob_c1), (ob_n0, ob_n1))

        rcopies = [pltpu.async_copy(wmt_h.at[w], row_m, sem_row),
                   pltpu.async_copy(wct_h.at[w], row_c, sem_row),
                   pltpu.async_copy(wnt_h.at[w], row_n, sem_row)]
        rd = pltpu.async_copy(wdt_h.at[w], row_d, sem_row)

        ins, outs = {}, {}
        for t in range(3):
            ins[(0, t)] = pltpu.async_copy(
                idx_hs[t].at[pl.ds(0, CH)], ibufs[t][0], sem_in)
        for cp in rcopies:
            cp.wait()
        for ck in range(NCHK):
            cur, nxt = ck % 2, (ck + 1) % 2
            if ck + 1 < NCHK:
                for t in range(3):
                    ins[(ck + 1, t)] = pltpu.async_copy(
                        idx_hs[t].at[pl.ds((ck + 1) * CH, CH)],
                        ibufs[t][nxt], sem_in)
            for t in range(3):
                ins[(ck, t)].wait()
            if ck >= 2:
                for t in range(3):
                    outs[(ck - 2, t)].wait()
            _gather_chunk(rows, [ib[cur] for ib in ibufs],
                          [ob[cur] for ob in obufs])
            for t in range(3):
                outs[(ck, t)] = pltpu.async_copy(
                    obufs[t][cur], out_hs[t].at[w, pl.ds(ck * CH, CH)],
                    sem_out)
        for ck in (NCHK - 2, NCHK - 1):
            for t in range(3):
                outs[(ck, t)].wait()

        rd.wait()
        for half in range(2):
            dr = w + half * NW
            dins, douts = {}, {}
            dins[0] = pltpu.async_copy(
                dn_h.at[pl.ds(0, CH)], ib_m0, sem_in)
            for ck in range(NCHK):
                cur, nxt = ck % 2, (ck + 1) % 2
                if ck + 1 < NCHK:
                    dins[ck + 1] = pltpu.async_copy(
                        dn_h.at[pl.ds((ck + 1) * CH, CH)],
                        (ib_m0, ib_m1)[nxt], sem_in)
                dins[ck].wait()
                if ck >= 2:
                    douts[ck - 2].wait()
                _gather_chunk((row_d,), [(ib_m0, ib_m1)[cur]],
                              [(ob_m0, ob_m1)[cur]])
                douts[ck] = pltpu.async_copy(
                    (ob_m0, ob_m1)[cur], odt_h.at[dr, pl.ds(ck * CH, CH)],
                    sem_out)
            for ck in (NCHK - 2, NCHK - 1):
                douts[ck].wait()
            if half == 0:
                pltpu.async_copy(wdt_h.at[w + NW], row_d, sem_row).wait()

    return gather4


_gather4 = _make_gather4()


LIN_BLK = 2048


def _lin_body(x1, x2, x3, x4, x5, w_ref, b_ref, o1, o2, o3, o4, o5):
    for k, (x, o) in enumerate(((x1, o1), (x2, o2), (x3, o3), (x4, o4),
                                (x5, o5))):
        o[...] = w_ref[k] * x[...][None, :] + b_ref[k]


def _lin5(xs, ws, bs):
    x_spec = pl.BlockSpec((LIN_BLK,), lambda i: (i,))
    wb_spec = pl.BlockSpec((5, D_LIN, 1), lambda i: (0, 0, 0))
    o_spec = pl.BlockSpec((D_LIN, LIN_BLK), lambda i: (0, i))
    w5 = jnp.stack([w.reshape(D_LIN) for w in ws])[:, :, None]
    b5 = jnp.stack([b.reshape(D_LIN) for b in bs])[:, :, None]
    outs = pl.pallas_call(
        _lin_body,
        grid=(B // LIN_BLK,),
        in_specs=[x_spec] * 5 + [wb_spec, wb_spec],
        out_specs=[o_spec] * 5,
        out_shape=[jax.ShapeDtypeStruct((D_LIN, B), jnp.float32)] * 5,
    )(*[x.reshape(B) for x in xs], w5, b5)
    return [o.T for o in outs]


def kernel(media_type, temperature, pre_culture_time, pre_culture_od600,
           drug_culture_time, drug_name, concentration, carbon_source,
           nitrogen_source, W_media, W_drug, W_carbon, W_nitrogen,
           W_temp, b_temp, W_pct, b_pct, W_od, b_od, W_dct, b_dct,
           W_conc, b_conc):
    mt = media_type.astype(jnp.int32)
    dn = drug_name.astype(jnp.int32)
    cs = carbon_source.astype(jnp.int32)
    ns_ = nitrogen_source.astype(jnp.int32)

    omt, odt, oct_, ont = _gather4(
        mt, dn, cs, ns_, W_media.T, W_drug.T, W_carbon.T, W_nitrogen.T)

    lt, lpct, lod, ldct, lconc = _lin5(
        (temperature, pre_culture_time, pre_culture_od600, drug_culture_time,
         concentration),
        (W_temp, W_pct, W_od, W_dct, W_conc),
        (b_temp, b_pct, b_od, b_dct, b_conc))

    return (omt.T, lt, lpct, lod, ldct, odt.T, lconc, oct_.T, ont.T)


# speedup vs baseline: 4.1486x; 1.0122x over previous
import functools

import jax
import jax.numpy as jnp
from jax import lax
from jax.experimental import pallas as pl
from jax.experimental.pallas import tpu as pltpu
from jax.experimental.pallas import tpu_sc as plsc

B = 16384
V_SMALL = 1000
V_DRUG = 100000
D_EMB = 32
D_DRUG = 64
D_LIN = 32

NC = 2
NS = 16
NW = NC * NS
CH = 2048
NCHK = B // CH
UNROLL = 8
GROUPS = CH // 16


def _zero_entry0(row_v):
    m = jnp.where(lax.iota(jnp.int32, 16) == 0, 0.0, 1.0)
    row_v[pl.ds(0, 16)] = row_v[pl.ds(0, 16)] * m


def _gather_chunk(rows, idxs, outs):
    def body(g, carry):
        base = g * (16 * UNROLL)
        for u in range(UNROLL):
            off = base + u * 16
            for row_v, idx_v, out_v in zip(rows, idxs, outs):
                idx16 = idx_v[pl.ds(off, 16)]
                out_v[pl.ds(off, 16)] = plsc.load_gather(row_v, [idx16])
        return carry

    lax.fori_loop(0, GROUPS // UNROLL, body, 0)


def _make_gather4():
    mesh = plsc.VectorSubcoreMesh(core_axis_name="c", subcore_axis_name="s",
                                  num_cores=NC, num_subcores=NS)

    @functools.partial(
        pl.kernel,
        mesh=mesh,
        compiler_params=pltpu.CompilerParams(needs_layout_passes=False),
        out_type=(
            jax.ShapeDtypeStruct((D_EMB, B), jnp.float32),
            jax.ShapeDtypeStruct((D_DRUG, B), jnp.float32),
            jax.ShapeDtypeStruct((D_EMB, B), jnp.float32),
            jax.ShapeDtypeStruct((D_EMB, B), jnp.float32),
        ),
        scratch_types=[
            pltpu.VMEM((V_SMALL,), jnp.float32),
            pltpu.VMEM((V_DRUG,), jnp.float32),
            pltpu.VMEM((V_SMALL,), jnp.float32),
            pltpu.VMEM((V_SMALL,), jnp.float32),
            pltpu.VMEM((CH,), jnp.int32),
            pltpu.VMEM((CH,), jnp.int32),
            pltpu.VMEM((CH,), jnp.int32),
            pltpu.VMEM((CH,), jnp.int32),
            pltpu.VMEM((CH,), jnp.int32),
            pltpu.VMEM((CH,), jnp.int32),
            pltpu.VMEM((CH,), jnp.float32),
            pltpu.VMEM((CH,), jnp.float32),
            pltpu.VMEM((CH,), jnp.float32),
            pltpu.VMEM((CH,), jnp.float32),
            pltpu.VMEM((CH,), jnp.float32),
            pltpu.VMEM((CH,), jnp.float32),
            pltpu.SemaphoreType.DMA,
            pltpu.SemaphoreType.DMA,
            pltpu.SemaphoreType.DMA,
        ],
    )
    def gather4(mt_h, dn_h, cs_h, ns_h, wmt_h, wdt_h, wct_h, wnt_h,
                omt_h, odt_h, oct_h, ont_h,
                row_m, row_d, row_c, row_n,
                ib_m0, ib_m1, ib_c0, ib_c1, ib_n0, ib_n1,
                ob_m0, ob_m1, ob_c0, ob_c1, ob_n0, ob_n1,
                sem_row, sem_in, sem_out):
        w = lax.axis_index("s") * NC + lax.axis_index("c")

        rows = (row_m, row_c, row_n)
        idx_hs = (mt_h, cs_h, ns_h)
        out_hs = (omt_h, oct_h, ont_h)
        ibufs = ((ib_m0, ib_m1), (ib_c0, ib_c1), (ib_n0, ib_n1))
        obufs = ((ob_m0, ob_m1), (ob_c0, ob_c1), (ob_n0, ob_n1))

        rcopies = [pltpu.async_copy(wmt_h.at[w], row_m, sem_row),
                   pltpu.async_copy(wct_h.at[w], row_c, sem_row),
                   pltpu.async_copy(wnt_h.at[w], row_n, sem_row)]
        rd = pltpu.async_copy(wdt_h.at[w], row_d, sem_row)

        ins, outs = {}, {}
        for t in range(3):
            ins[(0, t)] = pltpu.async_copy(
                idx_hs[t].at[pl.ds(0, CH)], ibufs[t][0], sem_in)
        for cp in rcopies:
            cp.wait()
        for row_v in rows:
            _zero_entry0(row_v)
        for ck in range(NCHK):
            cur, nxt = ck % 2, (ck + 1) % 2
            if ck + 1 < NCHK:
                for t in range(3):
                    ins[(ck + 1, t)] = pltpu.async_copy(
                        idx_hs[t].at[pl.ds((ck + 1) * CH, CH)],
                        ibufs[t][nxt], sem_in)
            for t in range(3):
                ins[(ck, t)].wait()
            if ck >= 2:
                for t in range(3):
                    outs[(ck - 2, t)].wait()
            _gather_chunk(rows, [ib[cur] for ib in ibufs],
                          [ob[cur] for ob in obufs])
            for t in range(3):
                outs[(ck, t)] = pltpu.async_copy(
                    obufs[t][cur], out_hs[t].at[w, pl.ds(ck * CH, CH)],
                    sem_out)
        for ck in (NCHK - 2, NCHK - 1):
            for t in range(3):
                outs[(ck, t)].wait()

        rd.wait()
        _zero_entry0(row_d)
        for half in range(2):
            dr = w + half * NW
            dins, douts = {}, {}
            dins[0] = pltpu.async_copy(
                dn_h.at[pl.ds(0, CH)], ib_m0, sem_in)
            for ck in range(NCHK):
                cur, nxt = ck % 2, (ck + 1) % 2
                if ck + 1 < NCHK:
                    dins[ck + 1] = pltpu.async_copy(
                        dn_h.at[pl.ds((ck + 1) * CH, CH)],
                        (ib_m0, ib_m1)[nxt], sem_in)
                dins[ck].wait()
                if ck >= 2:
                    douts[ck - 2].wait()
                _gather_chunk((row_d,), [(ib_m0, ib_m1)[cur]],
                              [(ob_m0, ob_m1)[cur]])
                douts[ck] = pltpu.async_copy(
                    (ob_m0, ob_m1)[cur], odt_h.at[dr, pl.ds(ck * CH, CH)],
                    sem_out)
            for ck in (NCHK - 2, NCHK - 1):
                douts[ck].wait()
            if half == 0:
                pltpu.async_copy(wdt_h.at[w + NW], row_d, sem_row).wait()
                _zero_entry0(row_d)

    return gather4


_gather4 = _make_gather4()


LIN_BLK = 2048


def _lin_body(x1, x2, x3, x4, x5, w_ref, b_ref, o1, o2, o3, o4, o5):
    for k, (x, o) in enumerate(((x1, o1), (x2, o2), (x3, o3), (x4, o4),
                                (x5, o5))):
        o[...] = w_ref[k] * x[...][None, :] + b_ref[k]


def _lin5(xs, ws, bs):
    x_spec = pl.BlockSpec((LIN_BLK,), lambda i: (i,))
    wb_spec = pl.BlockSpec((5, D_LIN, 1), lambda i: (0, 0, 0))
    o_spec = pl.BlockSpec((D_LIN, LIN_BLK), lambda i: (0, i))
    w5 = jnp.stack([w.reshape(D_LIN) for w in ws])[:, :, None]
    b5 = jnp.stack([b.reshape(D_LIN) for b in bs])[:, :, None]
    outs = pl.pallas_call(
        _lin_body,
        grid=(B // LIN_BLK,),
        in_specs=[x_spec] * 5 + [wb_spec, wb_spec],
        out_specs=[o_spec] * 5,
        out_shape=[jax.ShapeDtypeStruct((D_LIN, B), jnp.float32)] * 5,
    )(*[x.reshape(B) for x in xs], w5, b5)
    return [o.T for o in outs]


def kernel(media_type, temperature, pre_culture_time, pre_culture_od600,
           drug_culture_time, drug_name, concentration, carbon_source,
           nitrogen_source, W_media, W_drug, W_carbon, W_nitrogen,
           W_temp, b_temp, W_pct, b_pct, W_od, b_od, W_dct, b_dct,
           W_conc, b_conc):
    mt = media_type.astype(jnp.int32)
    dn = drug_name.astype(jnp.int32)
    cs = carbon_source.astype(jnp.int32)
    ns_ = nitrogen_source.astype(jnp.int32)

    omt, odt, oct_, ont = _gather4(
        mt, dn, cs, ns_, W_media.T, W_drug.T, W_carbon.T, W_nitrogen.T)

    lt, lpct, lod, ldct, lconc = _lin5(
        (temperature, pre_culture_time, pre_culture_od600, drug_culture_time,
         concentration),
        (W_temp, W_pct, W_od, W_dct, W_conc),
        (b_temp, b_pct, b_od, b_dct, b_conc))

    return (omt.T, lt, lpct, lod, ldct, odt.T, lconc, oct_.T, ont.T)


# speedup vs baseline: 4.8056x; 1.1584x over previous
import functools

import jax
import jax.numpy as jnp
from jax import lax
from jax.experimental import pallas as pl
from jax.experimental.pallas import tpu as pltpu
from jax.experimental.pallas import tpu_sc as plsc

B = 16384
V_SMALL = 1000
V_DRUG = 100000
D_EMB = 32
D_DRUG = 64
D_LIN = 32

NC = 2
NS = 16
NW = NC * NS
CH = 4096
NCHK = B // CH
NBUF = 3
UNROLL = 8
GROUPS = CH // 16


def _zero_entry0(row_v):
    m = jnp.where(lax.iota(jnp.int32, 16) == 0, 0.0, 1.0)
    row_v[pl.ds(0, 16)] = row_v[pl.ds(0, 16)] * m


def _gather_chunk(row_v, idx_v, out_v):
    def body(g, carry):
        base = g * (16 * UNROLL)
        for u in range(UNROLL):
            off = base + u * 16
            idx16 = idx_v[pl.ds(off, 16)]
            out_v[pl.ds(off, 16)] = plsc.load_gather(row_v, [idx16])
        return carry

    lax.fori_loop(0, GROUPS // UNROLL, body, 0)


def _make_gather4():
    mesh = plsc.VectorSubcoreMesh(core_axis_name="c", subcore_axis_name="s",
                                  num_cores=NC, num_subcores=NS)

    @functools.partial(
        pl.kernel,
        mesh=mesh,
        compiler_params=pltpu.CompilerParams(needs_layout_passes=False),
        out_type=(
            jax.ShapeDtypeStruct((D_EMB, B), jnp.float32),
            jax.ShapeDtypeStruct((D_DRUG, B), jnp.float32),
            jax.ShapeDtypeStruct((D_EMB, B), jnp.float32),
            jax.ShapeDtypeStruct((D_EMB, B), jnp.float32),
        ),
        scratch_types=[
            pltpu.VMEM((V_SMALL,), jnp.float32),
            pltpu.VMEM((V_DRUG,), jnp.float32),
            pltpu.VMEM((V_SMALL,), jnp.float32),
            pltpu.VMEM((V_SMALL,), jnp.float32),
            pltpu.VMEM((CH,), jnp.int32),
            pltpu.VMEM((CH,), jnp.int32),
            pltpu.VMEM((CH,), jnp.int32),
            pltpu.VMEM((CH,), jnp.float32),
            pltpu.VMEM((CH,), jnp.float32),
            pltpu.VMEM((CH,), jnp.float32),
            pltpu.SemaphoreType.DMA,
            pltpu.SemaphoreType.DMA,
            pltpu.SemaphoreType.DMA,
        ],
    )
    def gather4(mt_h, dn_h, cs_h, ns_h, wmt_h, wdt_h, wct_h, wnt_h,
                omt_h, odt_h, oct_h, ont_h,
                row_m, row_d, row_c, row_n,
                ib0, ib1, ib2, ob0, ob1, ob2,
                sem_row, sem_in, sem_out):
        w = lax.axis_index("s") * NC + lax.axis_index("c")
        ibufs = (ib0, ib1, ib2)
        obufs = (ob0, ob1, ob2)

        rm = pltpu.async_copy(wmt_h.at[w], row_m, sem_row)
        rd = pltpu.async_copy(wdt_h.at[w], row_d, sem_row)
        rc = pltpu.async_copy(wct_h.at[w], row_c, sem_row)
        rn = pltpu.async_copy(wnt_h.at[w], row_n, sem_row)

        passes = [
            (row_m, rm, mt_h, omt_h, w),
            (row_d, rd, dn_h, odt_h, w),
            (row_c, rc, cs_h, oct_h, w),
            (row_n, rn, ns_h, ont_h, w),
            (row_d, None, dn_h, odt_h, w + NW),
        ]
        steps = [(p, ck) for p in passes for ck in range(NCHK)]
        n = len(steps)

        pulls = [None] * n
        writes = [None] * n

        def pull(i):
            (_, _, idx_h, _, _), ck = steps[i]
            pulls[i] = pltpu.async_copy(
                idx_h.at[pl.ds(ck * CH, CH)], ibufs[i % NBUF], sem_in)

        pull(0)
        pull(1)
        rd2 = None
        for i in range(n):
            (row_v, rcopy, _, out_h, orow), ck = steps[i]
            if i + 2 < n:
                pull(i + 2)
            if ck == 0:
                if rcopy is not None:
                    rcopy.wait()
                else:
                    rd2.wait()
                _zero_entry0(row_v)
            pulls[i].wait()
            if i >= NBUF:
                writes[i - NBUF].wait()
            _gather_chunk(row_v, ibufs[i % NBUF], obufs[i % NBUF])
            if row_v is row_d and ck == NCHK - 1 and rd2 is None:
                rd2 = pltpu.async_copy(wdt_h.at[w + NW], row_d, sem_row)
            writes[i] = pltpu.async_copy(
                obufs[i % NBUF], out_h.at[orow, pl.ds(ck * CH, CH)], sem_out)
        for i in range(n - NBUF, n):
            writes[i].wait()

    return gather4


_gather4 = _make_gather4()


LIN_BLK = 2048


def _lin_body(x1, x2, x3, x4, x5, w_ref, b_ref, o1, o2, o3, o4, o5):
    for k, (x, o) in enumerate(((x1, o1), (x2, o2), (x3, o3), (x4, o4),
                                (x5, o5))):
        o[...] = w_ref[k] * x[...][None, :] + b_ref[k]


def _lin5(xs, ws, bs):
    x_spec = pl.BlockSpec((LIN_BLK,), lambda i: (i,))
    wb_spec = pl.BlockSpec((5, D_LIN, 1), lambda i: (0, 0, 0))
    o_spec = pl.BlockSpec((D_LIN, LIN_BLK), lambda i: (0, i))
    w5 = jnp.stack([w.reshape(D_LIN) for w in ws])[:, :, None]
    b5 = jnp.stack([b.reshape(D_LIN) for b in bs])[:, :, None]
    outs = pl.pallas_call(
        _lin_body,
        grid=(B // LIN_BLK,),
        in_specs=[x_spec] * 5 + [wb_spec, wb_spec],
        out_specs=[o_spec] * 5,
        out_shape=[jax.ShapeDtypeStruct((D_LIN, B), jnp.float32)] * 5,
    )(*[x.reshape(B) for x in xs], w5, b5)
    return [o.T for o in outs]


def kernel(media_type, temperature, pre_culture_time, pre_culture_od600,
           drug_culture_time, drug_name, concentration, carbon_source,
           nitrogen_source, W_media, W_drug, W_carbon, W_nitrogen,
           W_temp, b_temp, W_pct, b_pct, W_od, b_od, W_dct, b_dct,
           W_conc, b_conc):
    mt = media_type.astype(jnp.int32)
    dn = drug_name.astype(jnp.int32)
    cs = carbon_source.astype(jnp.int32)
    ns_ = nitrogen_source.astype(jnp.int32)

    omt, odt, oct_, ont = _gather4(
        mt, dn, cs, ns_, W_media.T, W_drug.T, W_carbon.T, W_nitrogen.T)

    lt, lpct, lod, ldct, lconc = _lin5(
        (temperature, pre_culture_time, pre_culture_od600, drug_culture_time,
         concentration),
        (W_temp, W_pct, W_od, W_dct, W_conc),
        (b_temp, b_pct, b_od, b_dct, b_conc))

    return (omt.T, lt, lpct, lod, ldct, odt.T, lconc, oct_.T, ont.T)
